# baseline (speedup 1.0000x reference)
import functools

import jax
import jax.numpy as jnp
from jax import lax
from jax.experimental import pallas as pl
from jax.experimental.pallas import tpu as pltpu
from jax.experimental.pallas import tpu_sc as plsc

N = 10000
D = 128
E = 320000
NC = 2
NS = 16
NW = NC * NS
K = 128
CHUNKS = 80
E_PAD = NW * CHUNKS * K
IB = 8
NP = 79 * K
RCHUNKS = 79
RC_PER_TILE = 8
CH0 = 160
CH1 = 0


def _prologue(idz_hbm, za_hbm, rows_v, idz_v, acc_sh, s):
    pltpu.sync_copy(idz_hbm.at[pl.ds(s * RC_PER_TILE, RC_PER_TILE)], idz_v)
    pltpu.sync_copy(za_hbm, rows_v)
    for t in range(RC_PER_TILE):
        pltpu.sync_copy(rows_v, acc_sh.at[idz_v.at[t]])
    plsc.subcore_barrier()


def _epilogue(out_hbm, rows_v, idz_v, acc_sh, sem, c, s):
    plsc.subcore_barrier()
    for t in range(RC_PER_TILE):
        rc = s * RC_PER_TILE + t

        @pl.when(rc < RCHUNKS)
        def _wb():
            pltpu.async_copy(acc_sh.at[idz_v.at[t]], rows_v, sem).wait()
            pltpu.sync_copy(rows_v, out_hbm.at[pl.ds(c * NP + rc * K, K)])


def _sc_agg_body(h_hbm, src_hbm, dst_hbm, idz_hbm, za_hbm, out_hbm,
                 src_v, dst_v, rows_v, rows2_v, idz_v, acc_sh,
                 gsem0, gsem1, ssem0, ssem1):
    c = lax.axis_index("c")
    s = lax.axis_index("s")

    _prologue(idz_hbm, za_hbm, rows_v, idz_v, acc_sh, s)

    bufs = (rows_v, rows2_v)
    gsems = (gsem0, gsem1)
    ssems = (ssem0, ssem1)

    base = jnp.where(c == 0, s * CH0, NS * CH0 + s * CH1)
    ngroups = jnp.where(c == 0, CH0 // IB, CH1 // IB)

    @pl.loop(0, ngroups)
    def _group(g):
        pltpu.sync_copy(src_hbm.at[pl.ds(base + g * IB, IB)], src_v)
        pltpu.sync_copy(dst_hbm.at[pl.ds(base + g * IB, IB)], dst_v)
        gd = [None] * IB
        sd = [None] * IB
        gd[0] = pltpu.async_copy(h_hbm.at[src_v.at[0]], bufs[0], gsems[0])
        for j in range(IB):
            cb, nb = j % 2, (j + 1) % 2
            if j + 1 < IB:
                if j >= 1:
                    sd[j - 1].wait()
                gd[j + 1] = pltpu.async_copy(
                    h_hbm.at[src_v.at[j + 1]], bufs[nb], gsems[nb])
            gd[j].wait()
            sd[j] = pltpu.async_copy(
                bufs[cb], acc_sh.at[dst_v.at[j]], ssems[cb], add=True)
        sd[IB - 2].wait()
        sd[IB - 1].wait()

    _epilogue(out_hbm, rows_v, idz_v, acc_sh, gsem0, c, s)


def _sc_deg_body(dst_hbm, idz_hbm, za_hbm, ones_hbm, out_hbm,
                 dst_v, rows_v, idz_v, acc_sh, sem, sem2):
    c = lax.axis_index("c")
    s = lax.axis_index("s")
    wid = s * NC + c

    _prologue(idz_hbm, za_hbm, rows_v, idz_v, acc_sh, s)
    pltpu.sync_copy(ones_hbm, rows_v)

    @pl.loop(0, CHUNKS // IB)
    def _group(g):
        pltpu.sync_copy(dst_hbm.at[pl.ds(wid * CHUNKS + g * IB, IB)], dst_v)
        sd = [None] * IB
        for j in range(IB):
            if j >= 2:
                sd[j - 2].wait()
            sd[j] = pltpu.async_copy(
                rows_v, acc_sh.at[dst_v.at[j]], sem2 if j % 2 else sem,
                add=True)
        sd[IB - 2].wait()
        sd[IB - 1].wait()

    _epilogue(out_hbm, rows_v, idz_v, acc_sh, sem, c, s)


def _make_sc(body, with_src):
    mesh = plsc.VectorSubcoreMesh(core_axis_name="c", subcore_axis_name="s")
    if with_src:
        scratch = [
            pltpu.VMEM((IB, K), jnp.int32),
            pltpu.VMEM((IB, K), jnp.int32),
            pltpu.VMEM((K, D), jnp.float32),
            pltpu.VMEM((K, D), jnp.float32),
            pltpu.VMEM((RC_PER_TILE, K), jnp.int32),
            pltpu.VMEM_SHARED((NP, D), jnp.float32),
            pltpu.SemaphoreType.DMA,
            pltpu.SemaphoreType.DMA,
            pltpu.SemaphoreType.DMA,
            pltpu.SemaphoreType.DMA,
        ]
    else:
        scratch = [
            pltpu.VMEM((IB, K), jnp.int32),
            pltpu.VMEM((K, D), jnp.float32),
            pltpu.VMEM((RC_PER_TILE, K), jnp.int32),
            pltpu.VMEM_SHARED((NP, D), jnp.float32),
            pltpu.SemaphoreType.DMA,
            pltpu.SemaphoreType.DMA,
        ]
    return pl.kernel(
        body,
        out_type=jax.ShapeDtypeStruct((NC * NP, D), jnp.float32),
        mesh=mesh,
        scratch_types=scratch,
    )


_sc_agg = _make_sc(_sc_agg_body, True)
_sc_deg = _make_sc(_sc_deg_body, False)


def _dense_body(relu, x_ref, agg_ref, deg_ref, ws_ref, wn_ref, b_ref, o_ref):
    deg = deg_ref[0, :, 0:1] + deg_ref[1, :, 0:1]
    mean = (agg_ref[0] + agg_ref[1]) / jnp.maximum(deg, 1.0)
    y = jnp.dot(x_ref[...], ws_ref[...], preferred_element_type=jnp.float32)
    y = y + jnp.dot(mean, wn_ref[...], preferred_element_type=jnp.float32)
    y = y + b_ref[...]
    if relu:
        y = jnp.maximum(y, 0.0)
    o_ref[...] = y


def _dense(x, agg, deg, w_self, w_neigh, b, relu):
    br = 1000
    return pl.pallas_call(
        functools.partial(_dense_body, relu),
        grid=(N // br,),
        in_specs=[
            pl.BlockSpec((br, D), lambda i: (i, 0)),
            pl.BlockSpec((NC, br, D), lambda i: (0, i, 0)),
            pl.BlockSpec((NC, br, D), lambda i: (0, i, 0)),
            pl.BlockSpec((D, D), lambda i: (0, 0)),
            pl.BlockSpec((D, D), lambda i: (0, 0)),
            pl.BlockSpec((1, D), lambda i: (0, 0)),
        ],
        out_specs=pl.BlockSpec((br, D), lambda i: (i, 0)),
        out_shape=jax.ShapeDtypeStruct((N, D), jnp.float32),
    )(x, agg, deg, w_self, w_neigh, b.reshape(1, D))


def kernel(x, edge_index, W1_self, W1_neigh, b1, W2_self, W2_neigh, b2):
    src = edge_index[0].astype(jnp.int32)
    dst = edge_index[1].astype(jnp.int32)
    pad = E_PAD - E
    src2 = jnp.concatenate([src, jnp.zeros((pad,), jnp.int32)]).reshape(NW * CHUNKS, K)
    dst2 = jnp.concatenate([dst, jnp.full((pad,), N, jnp.int32)]).reshape(NW * CHUNKS, K)
    idz = jnp.minimum(
        jnp.arange(NS * RC_PER_TILE * K, dtype=jnp.int32), jnp.int32(NP - 1)
    ).reshape(NS * RC_PER_TILE, K)
    za = jnp.zeros((K, D), jnp.float32)
    ones = jnp.ones((K, D), jnp.float32)

    deg = _sc_deg(dst2, idz, za, ones).reshape(NC, NP, D)
    agg1 = _sc_agg(x, src2, dst2, idz, za).reshape(NC, NP, D)
    h = _dense(x, agg1, deg, W1_self, W1_neigh, b1, relu=True)
    agg2 = _sc_agg(h, src2, dst2, idz, za).reshape(NC, NP, D)
    out = _dense(h, agg2, deg, W2_self, W2_neigh, b2, relu=False)
    return out


# speedup vs baseline: 1.1967x; 1.1967x over previous
import functools

import jax
import jax.numpy as jnp
from jax import lax
from jax.experimental import pallas as pl
from jax.experimental.pallas import tpu as pltpu
from jax.experimental.pallas import tpu_sc as plsc

N = 10000
D = 128
E = 320000
NC = 2
NS = 16
NW = NC * NS
K = 128
CHUNKS = 80
E_PAD = NW * CHUNKS * K
IB = 8
NP = 79 * K
RCHUNKS = 79
RC_PER_TILE = 8
CH0 = 96
CH1 = 64


def _prologue(idz_hbm, za_hbm, rows_v, idz_v, acc_sh, s):
    pltpu.sync_copy(idz_hbm.at[pl.ds(s * RC_PER_TILE, RC_PER_TILE)], idz_v)
    pltpu.sync_copy(za_hbm, rows_v)
    for t in range(RC_PER_TILE):
        pltpu.sync_copy(rows_v, acc_sh.at[idz_v.at[t]])
    plsc.subcore_barrier()


def _epilogue(out_hbm, rows_v, idz_v, acc_sh, sem, c, s):
    plsc.subcore_barrier()
    for t in range(RC_PER_TILE):
        rc = s * RC_PER_TILE + t

        @pl.when(rc < RCHUNKS)
        def _wb():
            pltpu.async_copy(acc_sh.at[idz_v.at[t]], rows_v, sem).wait()
            pltpu.sync_copy(rows_v, out_hbm.at[pl.ds(c * NP + rc * K, K)])


def _sc_agg_body(h_hbm, src_hbm, dst_hbm, idz_hbm, za_hbm, out_hbm,
                 src_v, dst_v, rows_v, rows2_v, idz_v, acc_sh,
                 gsem0, gsem1, ssem0, ssem1):
    c = lax.axis_index("c")
    s = lax.axis_index("s")

    _prologue(idz_hbm, za_hbm, rows_v, idz_v, acc_sh, s)

    bufs = (rows_v, rows2_v)
    gsems = (gsem0, gsem1)
    ssems = (ssem0, ssem1)

    base = jnp.where(c == 0, s * CH0, NS * CH0 + s * CH1)
    ngroups = jnp.where(c == 0, CH0 // IB, CH1 // IB)

    @pl.loop(0, ngroups)
    def _group(g):
        pltpu.sync_copy(src_hbm.at[pl.ds(base + g * IB, IB)], src_v)
        pltpu.sync_copy(dst_hbm.at[pl.ds(base + g * IB, IB)], dst_v)
        gd = [None] * IB
        sd = [None] * IB
        gd[0] = pltpu.async_copy(h_hbm.at[src_v.at[0]], bufs[0], gsems[0])
        for j in range(IB):
            cb, nb = j % 2, (j + 1) % 2
            if j + 1 < IB:
                if j >= 1:
                    sd[j - 1].wait()
                gd[j + 1] = pltpu.async_copy(
                    h_hbm.at[src_v.at[j + 1]], bufs[nb], gsems[nb])
            gd[j].wait()
            sd[j] = pltpu.async_copy(
                bufs[cb], acc_sh.at[dst_v.at[j]], ssems[cb], add=True)
        sd[IB - 2].wait()
        sd[IB - 1].wait()

    _epilogue(out_hbm, rows_v, idz_v, acc_sh, gsem0, c, s)


def _sc_deg_body(dst_hbm, idz_hbm, za_hbm, ones_hbm, out_hbm,
                 dst_v, rows_v, idz_v, acc_sh, sem, sem2):
    c = lax.axis_index("c")
    s = lax.axis_index("s")
    wid = s * NC + c

    _prologue(idz_hbm, za_hbm, rows_v, idz_v, acc_sh, s)
    pltpu.sync_copy(ones_hbm, rows_v)

    @pl.loop(0, CHUNKS // IB)
    def _group(g):
        pltpu.sync_copy(dst_hbm.at[pl.ds(wid * CHUNKS + g * IB, IB)], dst_v)
        sd = [None] * IB
        for j in range(IB):
            if j >= 2:
                sd[j - 2].wait()
            sd[j] = pltpu.async_copy(
                rows_v, acc_sh.at[dst_v.at[j]], sem2 if j % 2 else sem,
                add=True)
        sd[IB - 2].wait()
        sd[IB - 1].wait()

    _epilogue(out_hbm, rows_v, idz_v, acc_sh, sem, c, s)


def _make_sc(body, with_src):
    mesh = plsc.VectorSubcoreMesh(core_axis_name="c", subcore_axis_name="s")
    if with_src:
        scratch = [
            pltpu.VMEM((IB, K), jnp.int32),
            pltpu.VMEM((IB, K), jnp.int32),
            pltpu.VMEM((K, D), jnp.float32),
            pltpu.VMEM((K, D), jnp.float32),
            pltpu.VMEM((RC_PER_TILE, K), jnp.int32),
            pltpu.VMEM_SHARED((NP, D), jnp.float32),
            pltpu.SemaphoreType.DMA,
            pltpu.SemaphoreType.DMA,
            pltpu.SemaphoreType.DMA,
            pltpu.SemaphoreType.DMA,
        ]
    else:
        scratch = [
            pltpu.VMEM((IB, K), jnp.int32),
            pltpu.VMEM((K, D), jnp.float32),
            pltpu.VMEM((RC_PER_TILE, K), jnp.int32),
            pltpu.VMEM_SHARED((NP, D), jnp.float32),
            pltpu.SemaphoreType.DMA,
            pltpu.SemaphoreType.DMA,
        ]
    return pl.kernel(
        body,
        out_type=jax.ShapeDtypeStruct((NC * NP, D), jnp.float32),
        mesh=mesh,
        scratch_types=scratch,
    )


_sc_agg = _make_sc(_sc_agg_body, True)
_sc_deg = _make_sc(_sc_deg_body, False)


def _dense_body(relu, x_ref, agg_ref, deg_ref, ws_ref, wn_ref, b_ref, o_ref):
    deg = deg_ref[0, :, 0:1] + deg_ref[1, :, 0:1]
    mean = (agg_ref[0] + agg_ref[1]) / jnp.maximum(deg, 1.0)
    y = jnp.dot(x_ref[...], ws_ref[...], preferred_element_type=jnp.float32)
    y = y + jnp.dot(mean, wn_ref[...], preferred_element_type=jnp.float32)
    y = y + b_ref[...]
    if relu:
        y = jnp.maximum(y, 0.0)
    o_ref[...] = y


def _dense(x, agg, deg, w_self, w_neigh, b, relu):
    br = 1000
    return pl.pallas_call(
        functools.partial(_dense_body, relu),
        grid=(N // br,),
        in_specs=[
            pl.BlockSpec((br, D), lambda i: (i, 0)),
            pl.BlockSpec((NC, br, D), lambda i: (0, i, 0)),
            pl.BlockSpec((NC, br, D), lambda i: (0, i, 0)),
            pl.BlockSpec((D, D), lambda i: (0, 0)),
            pl.BlockSpec((D, D), lambda i: (0, 0)),
            pl.BlockSpec((1, D), lambda i: (0, 0)),
        ],
        out_specs=pl.BlockSpec((br, D), lambda i: (i, 0)),
        out_shape=jax.ShapeDtypeStruct((N, D), jnp.float32),
    )(x, agg, deg, w_self, w_neigh, b.reshape(1, D))


def kernel(x, edge_index, W1_self, W1_neigh, b1, W2_self, W2_neigh, b2):
    src = edge_index[0].astype(jnp.int32)
    dst = edge_index[1].astype(jnp.int32)
    pad = E_PAD - E
    src2 = jnp.concatenate([src, jnp.zeros((pad,), jnp.int32)]).reshape(NW * CHUNKS, K)
    dst2 = jnp.concatenate([dst, jnp.full((pad,), N, jnp.int32)]).reshape(NW * CHUNKS, K)
    idz = jnp.minimum(
        jnp.arange(NS * RC_PER_TILE * K, dtype=jnp.int32), jnp.int32(NP - 1)
    ).reshape(NS * RC_PER_TILE, K)
    za = jnp.zeros((K, D), jnp.float32)
    ones = jnp.ones((K, D), jnp.float32)

    deg = _sc_deg(dst2, idz, za, ones).reshape(NC, NP, D)
    agg1 = _sc_agg(x, src2, dst2, idz, za).reshape(NC, NP, D)
    h = _dense(x, agg1, deg, W1_self, W1_neigh, b1, relu=True)
    agg2 = _sc_agg(h, src2, dst2, idz, za).reshape(NC, NP, D)
    out = _dense(h, agg2, deg, W2_self, W2_neigh, b2, relu=False)
    return out


# speedup vs baseline: 1.2801x; 1.0697x over previous
import functools

import jax
import jax.numpy as jnp
from jax import lax
from jax.experimental import pallas as pl
from jax.experimental.pallas import tpu as pltpu
from jax.experimental.pallas import tpu_sc as plsc

N = 10000
D = 128
E = 320000
NC = 2
NS = 16
NW = NC * NS
K = 128
CHUNKS = 80
E_PAD = NW * CHUNKS * K
IB = 8
NP = 79 * K
RCHUNKS = 79
RC_PER_TILE = 8
CH0 = 128
CH1 = 32


def _prologue(idz_hbm, za_hbm, rows_v, idz_v, acc_sh, s):
    pltpu.sync_copy(idz_hbm.at[pl.ds(s * RC_PER_TILE, RC_PER_TILE)], idz_v)
    pltpu.sync_copy(za_hbm, rows_v)
    for t in range(RC_PER_TILE):
        pltpu.sync_copy(rows_v, acc_sh.at[idz_v.at[t]])
    plsc.subcore_barrier()


def _epilogue(out_hbm, rows_v, idz_v, acc_sh, sem, c, s):
    plsc.subcore_barrier()
    for t in range(RC_PER_TILE):
        rc = s * RC_PER_TILE + t

        @pl.when(rc < RCHUNKS)
        def _wb():
            pltpu.async_copy(acc_sh.at[idz_v.at[t]], rows_v, sem).wait()
            pltpu.sync_copy(rows_v, out_hbm.at[pl.ds(c * NP + rc * K, K)])


def _sc_agg_body(h_hbm, src_hbm, dst_hbm, idz_hbm, za_hbm, out_hbm,
                 src_v, dst_v, rows_v, rows2_v, idz_v, acc_sh,
                 gsem0, gsem1, ssem0, ssem1):
    c = lax.axis_index("c")
    s = lax.axis_index("s")

    _prologue(idz_hbm, za_hbm, rows_v, idz_v, acc_sh, s)

    bufs = (rows_v, rows2_v)
    gsems = (gsem0, gsem1)
    ssems = (ssem0, ssem1)

    base = jnp.where(c == 0, s * CH0, NS * CH0 + s * CH1)
    ngroups = jnp.where(c == 0, CH0 // IB, CH1 // IB)

    @pl.loop(0, ngroups)
    def _group(g):
        pltpu.sync_copy(src_hbm.at[pl.ds(base + g * IB, IB)], src_v)
        pltpu.sync_copy(dst_hbm.at[pl.ds(base + g * IB, IB)], dst_v)
        gd = [None] * IB
        sd = [None] * IB
        gd[0] = pltpu.async_copy(h_hbm.at[src_v.at[0]], bufs[0], gsems[0])
        for j in range(IB):
            cb, nb = j % 2, (j + 1) % 2
            if j + 1 < IB:
                if j >= 1:
                    sd[j - 1].wait()
                gd[j + 1] = pltpu.async_copy(
                    h_hbm.at[src_v.at[j + 1]], bufs[nb], gsems[nb])
            gd[j].wait()
            sd[j] = pltpu.async_copy(
                bufs[cb], acc_sh.at[dst_v.at[j]], ssems[cb], add=True)
        sd[IB - 2].wait()
        sd[IB - 1].wait()

    _epilogue(out_hbm, rows_v, idz_v, acc_sh, gsem0, c, s)


def _sc_deg_body(dst_hbm, idz_hbm, za_hbm, ones_hbm, out_hbm,
                 dst_v, rows_v, idz_v, acc_sh, sem, sem2):
    c = lax.axis_index("c")
    s = lax.axis_index("s")
    wid = s * NC + c

    _prologue(idz_hbm, za_hbm, rows_v, idz_v, acc_sh, s)
    pltpu.sync_copy(ones_hbm, rows_v)

    @pl.loop(0, CHUNKS // IB)
    def _group(g):
        pltpu.sync_copy(dst_hbm.at[pl.ds(wid * CHUNKS + g * IB, IB)], dst_v)
        sd = [None] * IB
        for j in range(IB):
            if j >= 2:
                sd[j - 2].wait()
            sd[j] = pltpu.async_copy(
                rows_v, acc_sh.at[dst_v.at[j]], sem2 if j % 2 else sem,
                add=True)
        sd[IB - 2].wait()
        sd[IB - 1].wait()

    _epilogue(out_hbm, rows_v, idz_v, acc_sh, sem, c, s)


def _make_sc(body, with_src):
    mesh = plsc.VectorSubcoreMesh(core_axis_name="c", subcore_axis_name="s")
    if with_src:
        scratch = [
            pltpu.VMEM((IB, K), jnp.int32),
            pltpu.VMEM((IB, K), jnp.int32),
            pltpu.VMEM((K, D), jnp.float32),
            pltpu.VMEM((K, D), jnp.float32),
            pltpu.VMEM((RC_PER_TILE, K), jnp.int32),
            pltpu.VMEM_SHARED((NP, D), jnp.float32),
            pltpu.SemaphoreType.DMA,
            pltpu.SemaphoreType.DMA,
            pltpu.SemaphoreType.DMA,
            pltpu.SemaphoreType.DMA,
        ]
    else:
        scratch = [
            pltpu.VMEM((IB, K), jnp.int32),
            pltpu.VMEM((K, D), jnp.float32),
            pltpu.VMEM((RC_PER_TILE, K), jnp.int32),
            pltpu.VMEM_SHARED((NP, D), jnp.float32),
            pltpu.SemaphoreType.DMA,
            pltpu.SemaphoreType.DMA,
        ]
    return pl.kernel(
        body,
        out_type=jax.ShapeDtypeStruct((NC * NP, D), jnp.float32),
        mesh=mesh,
        scratch_types=scratch,
    )


_sc_agg = _make_sc(_sc_agg_body, True)
_sc_deg = _make_sc(_sc_deg_body, False)


def _dense_body(relu, x_ref, agg_ref, deg_ref, ws_ref, wn_ref, b_ref, o_ref):
    deg = deg_ref[0, :, 0:1] + deg_ref[1, :, 0:1]
    mean = (agg_ref[0] + agg_ref[1]) / jnp.maximum(deg, 1.0)
    y = jnp.dot(x_ref[...], ws_ref[...], preferred_element_type=jnp.float32)
    y = y + jnp.dot(mean, wn_ref[...], preferred_element_type=jnp.float32)
    y = y + b_ref[...]
    if relu:
        y = jnp.maximum(y, 0.0)
    o_ref[...] = y


def _dense(x, agg, deg, w_self, w_neigh, b, relu):
    br = 1000
    return pl.pallas_call(
        functools.partial(_dense_body, relu),
        grid=(N // br,),
        in_specs=[
            pl.BlockSpec((br, D), lambda i: (i, 0)),
            pl.BlockSpec((NC, br, D), lambda i: (0, i, 0)),
            pl.BlockSpec((NC, br, D), lambda i: (0, i, 0)),
            pl.BlockSpec((D, D), lambda i: (0, 0)),
            pl.BlockSpec((D, D), lambda i: (0, 0)),
            pl.BlockSpec((1, D), lambda i: (0, 0)),
        ],
        out_specs=pl.BlockSpec((br, D), lambda i: (i, 0)),
        out_shape=jax.ShapeDtypeStruct((N, D), jnp.float32),
    )(x, agg, deg, w_self, w_neigh, b.reshape(1, D))


def kernel(x, edge_index, W1_self, W1_neigh, b1, W2_self, W2_neigh, b2):
    src = edge_index[0].astype(jnp.int32)
    dst = edge_index[1].astype(jnp.int32)
    pad = E_PAD - E
    src2 = jnp.concatenate([src, jnp.zeros((pad,), jnp.int32)]).reshape(NW * CHUNKS, K)
    dst2 = jnp.concatenate([dst, jnp.full((pad,), N, jnp.int32)]).reshape(NW * CHUNKS, K)
    idz = jnp.minimum(
        jnp.arange(NS * RC_PER_TILE * K, dtype=jnp.int32), jnp.int32(NP - 1)
    ).reshape(NS * RC_PER_TILE, K)
    za = jnp.zeros((K, D), jnp.float32)
    ones = jnp.ones((K, D), jnp.float32)

    deg = _sc_deg(dst2, idz, za, ones).reshape(NC, NP, D)
    agg1 = _sc_agg(x, src2, dst2, idz, za).reshape(NC, NP, D)
    h = _dense(x, agg1, deg, W1_self, W1_neigh, b1, relu=True)
    agg2 = _sc_agg(h, src2, dst2, idz, za).reshape(NC, NP, D)
    out = _dense(h, agg2, deg, W2_self, W2_neigh, b2, relu=False)
    return out
